# baseline (speedup 1.0000x reference)
import functools

import jax
import jax.numpy as jnp
from jax.experimental import pallas as pl

_N_E = 1024
_E_DIM = 256
_BETA = 0.25
_BB = 8


def _vq_body(z_ref, cb_ref, zq_ref, idx_ref, loss_ref, *, scale):
    b = pl.program_id(0)
    cb = cb_ref[...]
    S = z_ref.shape[2]
    cb16 = cb.astype(jnp.bfloat16)
    cb16n2 = (-2.0 * cb).astype(jnp.bfloat16)

    c2 = jnp.sum(cb * cb, axis=1, keepdims=True)

    part = jnp.zeros((1, 1), jnp.float32)
    for j in range(z_ref.shape[0]):
        zb = z_ref[j]
        z2 = jnp.sum(zb * zb, axis=0, keepdims=True)
        mn2 = jax.lax.dot_general(
            cb16n2, zb.astype(jnp.bfloat16), (((1,), (0,)), ((), ())),
            preferred_element_type=jnp.float32)
        d = (z2 + c2) + mn2

        idx = jnp.argmin(d, axis=0).astype(jnp.int32)
        idx_ref[j, 0, :] = idx

        onehot = (jax.lax.broadcasted_iota(jnp.int32, (_N_E, S), 0)
                  == idx[None, :]).astype(jnp.bfloat16)
        zq = jax.lax.dot_general(
            cb16, onehot, (((0,), (0,)), ((), ())),
            preferred_element_type=jnp.float32)
        zq_ref[j] = zq

        dmin = jnp.min(d, axis=0)
        part = part + (jnp.sum(dmin) * scale).reshape(1, 1)

    @pl.when(b == 0)
    def _():
        loss_ref[...] = jnp.zeros((1, 1), jnp.float32)

    loss_ref[...] += part


def kernel(z, codebook):
    B, C, H, W = z.shape
    S = H * W
    z3 = z.reshape(B, C, S)
    scale = (1.0 + _BETA) / (B * C * S)

    zq3, idx3, loss = pl.pallas_call(
        functools.partial(_vq_body, scale=scale),
        grid=(B // _BB,),
        in_specs=[
            pl.BlockSpec((_BB, C, S), lambda b: (b, 0, 0)),
            pl.BlockSpec((_N_E, _E_DIM), lambda b: (0, 0)),
        ],
        out_specs=[
            pl.BlockSpec((_BB, C, S), lambda b: (b, 0, 0)),
            pl.BlockSpec((_BB, 1, S), lambda b: (b, 0, 0)),
            pl.BlockSpec((1, 1), lambda b: (0, 0)),
        ],
        out_shape=[
            jax.ShapeDtypeStruct((B, C, S), jnp.float32),
            jax.ShapeDtypeStruct((B, 1, S), jnp.int32),
            jax.ShapeDtypeStruct((1, 1), jnp.float32),
        ],
    )(z3, codebook)

    z_q_out = zq3.reshape(B, C, H, W)
    indices_out = idx3.reshape(B, 1, H, W)
    return (z_q_out, loss[0, 0], indices_out)


# speedup vs baseline: 1.0265x; 1.0265x over previous
import functools

import jax
import jax.numpy as jnp
from jax.experimental import pallas as pl

_N_E = 1024
_E_DIM = 256
_BETA = 0.25
_BB = 4


def _vq_body(z_ref, cb_ref, zq_ref, idx_ref, loss_ref, *, scale):
    b = pl.program_id(0)
    cb = cb_ref[...]
    S = z_ref.shape[2]
    cb16 = cb.astype(jnp.bfloat16)
    cb16n2 = (-2.0 * cb).astype(jnp.bfloat16)

    c2 = jnp.sum(cb * cb, axis=1, keepdims=True)

    part = jnp.zeros((1, 1), jnp.float32)
    for j in range(z_ref.shape[0]):
        zb = z_ref[j]
        z2 = jnp.sum(zb * zb, axis=0, keepdims=True)
        mn2 = jax.lax.dot_general(
            cb16n2, zb.astype(jnp.bfloat16), (((1,), (0,)), ((), ())),
            preferred_element_type=jnp.float32)
        d = (z2 + c2) + mn2

        idx = jnp.argmin(d, axis=0).astype(jnp.int32)
        idx_ref[j, 0, :] = idx

        onehot = (jax.lax.broadcasted_iota(jnp.int32, (_N_E, S), 0)
                  == idx[None, :]).astype(jnp.bfloat16)
        zq = jax.lax.dot_general(
            cb16, onehot, (((0,), (0,)), ((), ())),
            preferred_element_type=jnp.float32)
        zq_ref[j] = zq

        dmin = jnp.min(d, axis=0)
        part = part + (jnp.sum(dmin) * scale).reshape(1, 1)

    @pl.when(b == 0)
    def _():
        loss_ref[...] = jnp.zeros((1, 1), jnp.float32)

    loss_ref[...] += part


def kernel(z, codebook):
    B, C, H, W = z.shape
    S = H * W
    z3 = z.reshape(B, C, S)
    scale = (1.0 + _BETA) / (B * C * S)

    zq3, idx3, loss = pl.pallas_call(
        functools.partial(_vq_body, scale=scale),
        grid=(B // _BB,),
        in_specs=[
            pl.BlockSpec((_BB, C, S), lambda b: (b, 0, 0)),
            pl.BlockSpec((_N_E, _E_DIM), lambda b: (0, 0)),
        ],
        out_specs=[
            pl.BlockSpec((_BB, C, S), lambda b: (b, 0, 0)),
            pl.BlockSpec((_BB, 1, S), lambda b: (b, 0, 0)),
            pl.BlockSpec((1, 1), lambda b: (0, 0)),
        ],
        out_shape=[
            jax.ShapeDtypeStruct((B, C, S), jnp.float32),
            jax.ShapeDtypeStruct((B, 1, S), jnp.int32),
            jax.ShapeDtypeStruct((1, 1), jnp.float32),
        ],
    )(z3, codebook)

    z_q_out = zq3.reshape(B, C, H, W)
    indices_out = idx3.reshape(B, 1, H, W)
    return (z_q_out, loss[0, 0], indices_out)
